# baseline (speedup 1.0000x reference)
import functools

import jax
import jax.numpy as jnp
from jax.experimental import pallas as pl
from jax.experimental.pallas import tpu as pltpu


def _diag_loss_kernel(x_hbm, out_ref, buf, sems, *, batch, scale):
    cw = buf.shape[-1]
    h = buf.shape[1]
    halves = [(0, h // 2), (h // 2, h - h // 2)] if h > 1 else [(0, h)]

    def window_copy(b, i):
        off, size = halves[i]
        return pltpu.make_async_copy(
            x_hbm.at[b, pl.ds(off, size), :, pl.ds(0, cw)],
            buf.at[b, pl.ds(off, size)],
            sems.at[len(halves) * b + i])

    for b in range(batch):
        for i in range(len(halves)):
            window_copy(b, i).start()

    total = jnp.zeros((), jnp.float32)
    for b in range(batch):
        for i in range(len(halves)):
            window_copy(b, i).wait()
        blk = buf[b].astype(jnp.float32)
        s_c = jnp.sum(jnp.sum(blk, axis=0), axis=0, keepdims=True)
        lane = jax.lax.broadcasted_iota(jnp.int32, s_c.shape, 1)
        total = total + jnp.sum(jnp.where(lane == b, s_c, 0.0))

    out_ref[0, 0] = total * jnp.float32(scale)


def kernel(output):
    B, C, H, W = output.shape
    scale = -1.0 / float(H * W)

    x = jnp.transpose(output, (0, 2, 3, 1))

    cw = min(C, max(128, -(-B // 128) * 128))
    buf_bytes = B * H * W * cw * jnp.dtype(output.dtype).itemsize

    loss = pl.pallas_call(
        functools.partial(_diag_loss_kernel, batch=B, scale=scale),
        out_shape=jax.ShapeDtypeStruct((1, 1), jnp.float32),
        in_specs=[pl.BlockSpec(memory_space=pl.ANY)],
        out_specs=pl.BlockSpec(memory_space=pltpu.SMEM),
        scratch_shapes=[
            pltpu.VMEM((B, H, W, cw), output.dtype),
            pltpu.SemaphoreType.DMA((2 * B,)),
        ],
        compiler_params=pltpu.CompilerParams(
            vmem_limit_bytes=buf_bytes + 8 * 1024 * 1024),
    )(x)
    return loss[0, 0]


# speedup vs baseline: 1.0149x; 1.0149x over previous
import functools

import jax
import jax.numpy as jnp
from jax.experimental import pallas as pl
from jax.experimental.pallas import tpu as pltpu


def _diag_loss_kernel(x_hbm, out_ref, buf, sems, *, batch, scale):
    cw = buf.shape[-1]

    def window_copy(b):
        return pltpu.make_async_copy(
            x_hbm.at[b, :, :, pl.ds(0, cw)], buf.at[b], sems.at[b])

    for b in range(batch):
        window_copy(b).start()

    total = jnp.zeros((), jnp.float32)
    for b in range(batch):
        window_copy(b).wait()
        blk = buf[b].astype(jnp.float32)
        s_c = jnp.sum(jnp.sum(blk, axis=0), axis=0, keepdims=True)
        lane = jax.lax.broadcasted_iota(jnp.int32, s_c.shape, 1)
        total = total + jnp.sum(jnp.where(lane == b, s_c, 0.0))

    out_ref[0, 0] = total * jnp.float32(scale)


def kernel(output):
    B, C, H, W = output.shape
    scale = -1.0 / float(H * W)

    x = jnp.transpose(output, (0, 2, 3, 1))

    cw = min(C, max(128, -(-B // 128) * 128))
    buf_bytes = B * H * W * cw * jnp.dtype(output.dtype).itemsize

    loss = pl.pallas_call(
        functools.partial(_diag_loss_kernel, batch=B, scale=scale),
        out_shape=jax.ShapeDtypeStruct((1, 1), jnp.float32),
        in_specs=[pl.BlockSpec(memory_space=pl.ANY)],
        out_specs=pl.BlockSpec(memory_space=pltpu.SMEM),
        scratch_shapes=[
            pltpu.VMEM((B, H, W, cw), output.dtype),
            pltpu.SemaphoreType.DMA((B,)),
        ],
        compiler_params=pltpu.CompilerParams(
            vmem_limit_bytes=buf_bytes + 8 * 1024 * 1024),
    )(x)
    return loss[0, 0]
